# baseline (speedup 1.0000x reference)
import functools

import jax
import jax.numpy as jnp
from jax import lax
from jax.experimental import pallas as pl
from jax.experimental.pallas import tpu as pltpu
from jax.experimental.pallas import tpu_sc as plsc

SEQ = 200
BATCH = 4096
DIM = 64

NC = 2
NS = 16
NW = NC * NS
BT = 128
NBT = BATCH // BT
NCHUNK = SEQ * NBT
CPW = NCHUNK // NW
TP = 129

_MESH = plsc.VectorSubcoreMesh(
    core_axis_name="c", subcore_axis_name="s", num_cores=NC, num_subcores=NS
)


@functools.partial(
    pl.kernel,
    out_type=jax.ShapeDtypeStruct((SEQ, DIM // 8, NBT, 8, BT), jnp.float32),
    mesh=_MESH,
    compiler_params=pltpu.CompilerParams(
        use_tc_tiling_on_sc=False, needs_layout_passes=False
    ),
    scratch_types=[
        pltpu.VMEM((CPW, BT), jnp.int32),
    ] + [pltpu.VMEM((BT, DIM), jnp.float32) for _ in range(5)]
      + [pltpu.VMEM((8, 8, TP), jnp.float32) for _ in range(5)]
      + [pltpu.SemaphoreType.DMA for _ in range(10)],
)
def _gather_kernel(table_hbm, idx_hbm, out_hbm, idx_v,
                   rows0, rows1, rows2, rows3, rows4, t0, t1, t2, t3, t4,
                   g0, g1, g2, g3, g4, w0, w1, w2, w3, w4):
    wid = lax.axis_index("s") * NC + lax.axis_index("c")
    base = wid * CPW

    pltpu.sync_copy(idx_hbm.at[pl.ds(base, CPW)], idx_v)

    bufs = ((rows0, t0, g0, w0), (rows1, t1, g1, w1),
            (rows2, t2, g2, w2), (rows3, t3, g3, w3),
            (rows4, t4, g4, w4))
    NBUF = len(bufs)

    lane = lax.iota(jnp.int32, 16)
    dtv = [(lane + 16 * gg) // 8 for gg in range(4)]
    drv = [(lane + 16 * gg) % 8 for gg in range(4)]

    def fire_gather(i, rows, sem):
        pltpu.async_copy(table_hbm.at[idx_v.at[i]], rows, sem)

    def out_block(i):
        c = base + i
        return out_hbm.at[c // NBT, :, c % NBT]

    def transpose(rows, tbuf):
        @plsc.parallel_loop(0, BT, unroll=4)
        def _bc(bc):
            bcv = lane * 0 + bc
            for gg in range(4):
                v = rows[bc, pl.ds(16 * gg, 16)]
                plsc.store_scatter(tbuf, [dtv[gg], drv[gg], bcv], v)

    for p, (rows, _, gsem, _) in enumerate(bufs):
        fire_gather(p, rows, gsem)

    @pl.loop(0, CPW, step=5)
    def _chunks(i):
        for p, (rows, tbuf, gsem, wsem) in enumerate(bufs):
            g = i + p
            pltpu.make_async_copy(table_hbm.at[idx_v.at[g]], rows, gsem).wait()

            @pl.when(g >= NBUF)
            def _():
                pltpu.make_async_copy(
                    tbuf.at[:, :, pl.ds(0, BT)], out_block(g - NBUF), wsem
                ).wait()

            transpose(rows, tbuf)
            pltpu.async_copy(tbuf.at[:, :, pl.ds(0, BT)], out_block(g), wsem)

            @pl.when(g + NBUF < CPW)
            def _():
                fire_gather(g + NBUF, rows, gsem)

    for p, (_, tbuf, _, wsem) in enumerate(bufs):
        pltpu.make_async_copy(
            tbuf.at[:, :, pl.ds(0, BT)], out_block(CPW - NBUF + p), wsem
        ).wait()


def kernel(input, table):
    idx = input.reshape(NCHUNK, BT)
    out5 = _gather_kernel(table, idx)
    return out5.transpose(0, 2, 4, 1, 3).reshape(SEQ, BATCH, DIM)
